# baseline (speedup 1.0000x reference)
import functools

import jax
import jax.numpy as jnp
from jax import lax
from jax.experimental import pallas as pl
from jax.experimental.pallas import tpu as pltpu
from jax.experimental.pallas import tpu_sc as plsc

H = 512
W = 512
WINDOW = 32
N_SEEDS = 256
NEIGH = 5
MIN_SEED = 0.5
MASK_THRESH = 0.5

NT = 32
CAP = 512
TOT = NT * CAP
ROWS_PER_TILE = H // NT
BAND = ROWS_PER_TILE * W
NPIX = H * W
MBITS = 0x7FFFFF

_mesh = plsc.VectorSubcoreMesh(
    core_axis_name="c", subcore_axis_name="s", num_cores=2, num_subcores=16)


def _wid():
    return lax.axis_index("s") * 2 + lax.axis_index("c")


def _lane():
    return lax.iota(jnp.int32, 16)


def _tc_body(fields_ref, seed_ref, emb_ref, keys_ref):
    f = fields_ref[...]
    step = jnp.float32(W * 64.0 / 256.0 / (W - 1))
    col = lax.broadcasted_iota(jnp.int32, (H, W), 1).astype(jnp.float32) * step
    row = lax.broadcasted_iota(jnp.int32, (H, W), 0).astype(jnp.float32) * step
    emb_ref[0] = jnp.tanh(f[0]) + col
    emb_ref[1] = jnp.tanh(f[1]) + row

    s = seed_ref[0]
    ninf = jnp.float32(-jnp.inf)
    rm = s
    for d in range(1, NEIGH + 1):
        left = jnp.concatenate([jnp.full((H, d), ninf), s[:, :-d]], axis=1)
        right = jnp.concatenate([s[:, d:], jnp.full((H, d), ninf)], axis=1)
        rm = jnp.maximum(rm, jnp.maximum(left, right))
    pm = rm
    for d in range(1, NEIGH + 1):
        up = jnp.concatenate([jnp.full((d, W), ninf), rm[:-d, :]], axis=0)
        dn = jnp.concatenate([rm[d:, :], jnp.full((d, W), ninf)], axis=0)
        pm = jnp.maximum(pm, jnp.maximum(up, dn))
    is_peak = (s == pm) & (s > MIN_SEED)
    keys_ref[...] = jnp.where(
        is_peak, lax.bitcast_convert_type(s, jnp.int32), jnp.int32(0))


def _tc_stage(fields, seed_map):
    return pl.pallas_call(
        _tc_body,
        out_shape=(
            jax.ShapeDtypeStruct((2, H, W), jnp.float32),
            jax.ShapeDtypeStruct((H, W), jnp.int32),
        ),
    )(fields, seed_map)


@functools.partial(
    pl.kernel,
    mesh=_mesh,
    compiler_params=pltpu.CompilerParams(needs_layout_passes=False),
    out_type=(
        jax.ShapeDtypeStruct((NT, CAP), jnp.int32),
        jax.ShapeDtypeStruct((NT, CAP), jnp.int32),
        jax.ShapeDtypeStruct((NT, 16), jnp.int32),
    ),
    scratch_types=[
        pltpu.VMEM((BAND,), jnp.int32),
        pltpu.VMEM((CAP,), jnp.int32),
        pltpu.VMEM((CAP,), jnp.int32),
        pltpu.VMEM((16,), jnp.int32),
    ],
)
def _sc_compact(keys_hbm, ck_hbm, ci_hbm, cnt_hbm, keys_v, ck_v, ci_v, cnt_v):
    wid = _wid()
    base = wid * BAND
    lane = _lane()
    pltpu.sync_copy(keys_hbm.at[pl.ds(base, BAND)], keys_v)

    def body(j, cnt_vec):
        kv = keys_v[pl.ds(j * 16, 16)]
        mask = kv > 0
        pos = cnt_vec + plsc.cumsum(mask.astype(jnp.int32)) - 1
        okm = mask & (pos < CAP)
        plsc.store_scatter(ck_v, [pos], kv, mask=okm)
        plsc.store_scatter(ci_v, [pos], base + j * 16 + lane, mask=okm)
        return cnt_vec + plsc.all_reduce_population_count(okm)

    cnt_vec = lax.fori_loop(0, BAND // 16, body, jnp.zeros((16,), jnp.int32))
    cnt_v[...] = cnt_vec
    pltpu.sync_copy(ck_v, ck_hbm.at[wid])
    pltpu.sync_copy(ci_v, ci_hbm.at[wid])
    pltpu.sync_copy(cnt_v, cnt_hbm.at[wid])


@functools.partial(
    pl.kernel,
    mesh=_mesh,
    compiler_params=pltpu.CompilerParams(needs_layout_passes=False),
    out_type=jax.ShapeDtypeStruct((N_SEEDS + 16, 16), jnp.float32),
    scratch_types=[
        pltpu.VMEM((NT, CAP), jnp.int32),
        pltpu.VMEM((NT, CAP), jnp.int32),
        pltpu.VMEM((NT, 16), jnp.int32),
        pltpu.VMEM((TOT,), jnp.int32),
        pltpu.VMEM((TOT,), jnp.int32),
        pltpu.VMEM((CAP,), jnp.int32),
        pltpu.VMEM((CAP,), jnp.int32),
        pltpu.VMEM((16, 16), jnp.float32),
        pltpu.VMEM((16,), jnp.float32),
        pltpu.VMEM((16,), jnp.float32),
        pltpu.VMEM((16,), jnp.float32),
        pltpu.VMEM((16,), jnp.float32),
        pltpu.SemaphoreType.DMA,
    ],
)
def _sc_select(ck_hbm, ci_hbm, cnt_hbm, emb_hbm, sig_hbm, seeds_hbm,
               ck_v, ci_v, cnt_v, gk, gi, wbi, wbr, rows, g0, g1, g2, g3, sem):
    wid = _wid()
    lane = _lane()
    zeros16 = jnp.zeros((16,), jnp.int32)

    pltpu.sync_copy(ck_hbm, ck_v)
    pltpu.sync_copy(ci_hbm, ci_v)
    pltpu.sync_copy(cnt_hbm, cnt_v)

    def outer(t, cnt_vec):
        c_t = cnt_v[t][0]

        def inner(j, cv):
            kv = ck_v[t, pl.ds(j * 16, 16)]
            iv = ci_v[t, pl.ds(j * 16, 16)]
            mask = (j * 16 + lane) < c_t
            pos = cv + plsc.cumsum(mask.astype(jnp.int32)) - 1
            plsc.store_scatter(gk, [pos], kv, mask=mask)
            plsc.store_scatter(gi, [pos], iv, mask=mask)
            return cv + plsc.all_reduce_population_count(mask)

        return lax.fori_loop(0, (c_t + 15) // 16, inner, cnt_vec)

    cnt_vec = lax.fori_loop(0, NT, outer, zeros16)
    n_total = cnt_vec[0]
    nv = (n_total + 15) // 16

    def count_ge(thr):
        def b(j, acc):
            kv = gk[pl.ds(j * 16, 16)]
            mm = kv & MBITS
            c = ((j * 16 + lane) < n_total) & (mm >= thr)
            return acc + plsc.all_reduce_population_count(c)

        return lax.fori_loop(0, nv, b, zeros16)[0]

    def bit_step(i, p):
        thr = p | lax.shift_left(jnp.int32(1), 22 - i)
        c1 = count_ge(thr)
        return jnp.where(c1 >= N_SEEDS, thr, p)

    thr_m = lax.fori_loop(0, 23, bit_step, jnp.int32(0))
    n_top = jnp.minimum(jnp.int32(N_SEEDS), n_total)

    def rank_of(mi, idxi):
        def rb(j, acc):
            kv = gk[pl.ds(j * 16, 16)]
            iv = gi[pl.ds(j * 16, 16)]
            mm = kv & MBITS
            hi = ((j * 16 + lane) < n_total) & (
                (mm > mi) | ((mm == mi) & (iv < idxi)))
            return acc + plsc.all_reduce_population_count(hi)

        return lax.fori_loop(0, nv, rb, zeros16)[0]

    def qbody(q, w):
        i = wid + q * NT
        live = i < n_total
        isafe = jnp.full((16,), jnp.minimum(i, TOT - 1), jnp.int32)
        mi = plsc.load_gather(gk, [isafe])[0] & MBITS
        idxi = plsc.load_gather(gi, [isafe])[0]
        live = live & (mi >= thr_m)
        r = lax.cond(live, lambda: rank_of(mi, idxi), lambda: jnp.int32(N_SEEDS))
        win = live & (r < N_SEEDS)
        lane0 = lane == 0
        plsc.store_scatter(wbi, [jnp.full((16,), w, jnp.int32)],
                           jnp.full((16,), idxi, jnp.int32), mask=lane0 & win)
        plsc.store_scatter(wbr, [jnp.full((16,), w, jnp.int32)],
                           jnp.full((16,), r, jnp.int32), mask=lane0 & win)
        return w + jnp.where(win, 1, 0)

    nq = (n_total - wid + NT - 1) // NT
    nq = jnp.maximum(nq, 0)
    w = lax.fori_loop(0, nq, qbody, jnp.int32(0))

    def zrow(j, _):
        plsc.store_scatter(rows, [jnp.full((16,), j, jnp.int32), lane],
                           jnp.zeros((16,), jnp.float32))
        return 0

    lax.fori_loop(0, 16, zrow, 0)
    pad0 = n_top + ((wid - n_top) % NT + NT) % NT

    def zfill(k, _):
        r = pad0 + k * NT

        @pl.when(r < N_SEEDS)
        def _():
            pltpu.sync_copy(rows.at[0], seeds_hbm.at[r])

        return 0

    lax.fori_loop(0, (N_SEEDS + NT - 1) // NT, zfill, 0)

    def chunk(c, _):
        iv = wbi[pl.ds(c * 16, 16)]
        lm = (c * 16 + lane) < w
        ivc = jnp.where(lm, iv, 0)
        pltpu.async_copy(emb_hbm.at[ivc], g0, sem).wait()
        pltpu.async_copy(emb_hbm.at[ivc + NPIX], g1, sem).wait()
        pltpu.async_copy(sig_hbm.at[ivc], g2, sem).wait()
        pltpu.async_copy(sig_hbm.at[ivc + NPIX], g3, sem).wait()

        def col(j):
            return [lane, jnp.full((16,), j, jnp.int32)]

        plsc.store_scatter(rows, col(0),
                           lax.shift_right_logical(ivc, 9).astype(jnp.float32))
        plsc.store_scatter(rows, col(1), (ivc & (W - 1)).astype(jnp.float32))
        plsc.store_scatter(rows, col(2), g0[...])
        plsc.store_scatter(rows, col(3), g1[...])
        plsc.store_scatter(rows, col(4), jnp.exp(g2[...] * 10.0))
        plsc.store_scatter(rows, col(5), jnp.exp(g3[...] * 10.0))
        plsc.store_scatter(rows, col(6), jnp.ones((16,), jnp.float32))

        def wr(k, _):
            pos = c * 16 + k

            @pl.when(pos < w)
            def _():
                rk = plsc.load_gather(
                    wbr, [jnp.full((16,), pos, jnp.int32)])[0]
                pltpu.sync_copy(rows.at[k], seeds_hbm.at[rk])

            return 0

        lax.fori_loop(0, 16, wr, 0)
        return 0

    lax.fori_loop(0, (w + 15) // 16, chunk, 0)


@functools.partial(
    pl.kernel,
    mesh=_mesh,
    compiler_params=pltpu.CompilerParams(needs_layout_passes=False),
    out_type=(
        jax.ShapeDtypeStruct((NPIX,), jnp.float32),
        jax.ShapeDtypeStruct((NPIX,), jnp.float32),
    ),
    scratch_types=[
        pltpu.VMEM((N_SEEDS, 16), jnp.float32),
        pltpu.VMEM((BAND + 128,), jnp.float32),
        pltpu.VMEM((BAND + 128,), jnp.float32),
        pltpu.VMEM((BAND + 128,), jnp.float32),
        pltpu.VMEM((BAND + 128,), jnp.float32),
        pltpu.SemaphoreType.DMA,
    ],
)
def _sc_paint(seeds_hbm, emb_hbm, lab_hbm, sco_hbm, seeds_v, e0v, e1v, mv, lv,
              sem):
    wid = _wid()
    lane = _lane()
    pltpu.sync_copy(seeds_hbm.at[pl.ds(0, N_SEEDS)], seeds_v)
    din = []
    for l in range(ROWS_PER_TILE):
        g = (wid + l * NT) * W
        din.append(pltpu.async_copy(
            emb_hbm.at[pl.ds(g, W)], e0v.at[pl.ds(l * W, W)], sem))
        din.append(pltpu.async_copy(
            emb_hbm.at[pl.ds(NPIX + g, W)], e1v.at[pl.ds(l * W, W)], sem))
    for d in din:
        d.wait()

    def zbody(j, _):
        z = jnp.zeros((16,), jnp.float32)
        mv[pl.ds(j * 16, 16)] = z
        lv[pl.ds(j * 16, 16)] = z
        return 0

    lax.fori_loop(0, BAND // 16, zbody, 0)

    def seed_body(e, _):
        rec = seeds_v[e]
        valid = rec[6]
        py = rec[0].astype(jnp.int32)
        px = rec[1].astype(jnp.int32)
        c0 = rec[2]
        c1 = rec[3]
        s0 = rec[4]
        s1 = rec[5]
        lab = (e + 1).astype(jnp.float32)
        ylo = jnp.maximum(py - WINDOW, 0)
        yhi = jnp.minimum(py + WINDOW, H)
        xlo = jnp.maximum(px - WINDOW, 0)
        xhi = jnp.minimum(px + WINDOW, W)
        y0 = ylo + jnp.mod(wid - ylo, NT)

        cb = lax.shift_right_logical(xlo, 4) * 16

        @pl.when(valid > 0.0)
        def _():
            def one_row(y):
                rowoff = ((y - wid) // NT) * W
                for cch in range(5):
                    base = rowoff + cb + cch * 16
                    bx = cb + cch * 16 + lane
                    mk = (bx >= xlo) & (bx < xhi)
                    a0 = e0v[pl.ds(base, 16)]
                    a1 = e1v[pl.ds(base, 16)]
                    pm = mv[pl.ds(base, 16)]
                    d0 = a0 - c0
                    d1 = a1 - c1
                    pr = jnp.exp(-(d0 * d0 * s0 + d1 * d1 * s1))
                    upd = mk & (pr >= pm)

                    @pl.when(jnp.any(upd))
                    def _():
                        plv = lv[pl.ds(base, 16)]
                        lc = jnp.where(pr >= MASK_THRESH, lab, 0.0)
                        nm = jnp.where(upd, jnp.maximum(pm, pr), pm)
                        nl = jnp.where(upd & (pr > pm), lc,
                                       jnp.where(upd & (pr == pm),
                                                 jnp.maximum(plv, lc), plv))
                        mv[pl.ds(base, 16)] = nm
                        lv[pl.ds(base, 16)] = nl

            for k in range(2):
                y = y0 + k * NT

                @pl.when(y < yhi)
                def _():
                    one_row(y)

        return 0

    lax.fori_loop(0, N_SEEDS, seed_body, 0)
    dout = []
    for l in range(ROWS_PER_TILE):
        g = (wid + l * NT) * W
        dout.append(pltpu.async_copy(
            lv.at[pl.ds(l * W, W)], lab_hbm.at[pl.ds(g, W)], sem))
        dout.append(pltpu.async_copy(
            mv.at[pl.ds(l * W, W)], sco_hbm.at[pl.ds(g, W)], sem))
    for d in dout:
        d.wait()


def kernel(fields, sigma, seed_map):
    emb, keys = _tc_stage(fields, seed_map)
    emb_flat = emb.reshape(-1)
    sig_flat = sigma.reshape(-1)
    ck, ci, cnt = _sc_compact(keys.reshape(-1))
    seeds = _sc_select(ck, ci, cnt, emb_flat, sig_flat)
    lab_flat, sco_flat = _sc_paint(seeds, emb_flat)
    return lab_flat.reshape(H, W), sco_flat.reshape(H, W)


# speedup vs baseline: 1.5378x; 1.5378x over previous
import functools

import jax
import jax.numpy as jnp
from jax import lax
from jax.experimental import pallas as pl
from jax.experimental.pallas import tpu as pltpu
from jax.experimental.pallas import tpu_sc as plsc

H = 512
W = 512
WINDOW = 32
N_SEEDS = 256
NEIGH = 5
MIN_SEED = 0.5
MASK_THRESH = 0.5

NT = 32
CAP = 512
TOT = NT * CAP
ROWS_PER_TILE = H // NT
BAND = ROWS_PER_TILE * W
NPIX = H * W
MBITS = 0x7FFFFF

_mesh = plsc.VectorSubcoreMesh(
    core_axis_name="c", subcore_axis_name="s", num_cores=2, num_subcores=16)


def _wid():
    return lax.axis_index("s") * 2 + lax.axis_index("c")


def _lane():
    return lax.iota(jnp.int32, 16)


def _tc_body(fields_ref, seed_ref, emb_ref, keys_ref):
    f = fields_ref[...]
    step = jnp.float32(W * 64.0 / 256.0 / (W - 1))
    col = lax.broadcasted_iota(jnp.int32, (H, W), 1).astype(jnp.float32) * step
    row = lax.broadcasted_iota(jnp.int32, (H, W), 0).astype(jnp.float32) * step
    emb_ref[0] = jnp.tanh(f[0]) + col
    emb_ref[1] = jnp.tanh(f[1]) + row

    s = seed_ref[0]
    ninf = jnp.float32(-jnp.inf)
    rm = s
    for d in range(1, NEIGH + 1):
        left = jnp.concatenate([jnp.full((H, d), ninf), s[:, :-d]], axis=1)
        right = jnp.concatenate([s[:, d:], jnp.full((H, d), ninf)], axis=1)
        rm = jnp.maximum(rm, jnp.maximum(left, right))
    pm = rm
    for d in range(1, NEIGH + 1):
        up = jnp.concatenate([jnp.full((d, W), ninf), rm[:-d, :]], axis=0)
        dn = jnp.concatenate([rm[d:, :], jnp.full((d, W), ninf)], axis=0)
        pm = jnp.maximum(pm, jnp.maximum(up, dn))
    is_peak = (s == pm) & (s > MIN_SEED)
    keys_ref[...] = jnp.where(
        is_peak, lax.bitcast_convert_type(s, jnp.int32), jnp.int32(0))


def _tc_stage(fields, seed_map):
    return pl.pallas_call(
        _tc_body,
        out_shape=(
            jax.ShapeDtypeStruct((2, H, W), jnp.float32),
            jax.ShapeDtypeStruct((H, W), jnp.int32),
        ),
    )(fields, seed_map)


@functools.partial(
    pl.kernel,
    mesh=_mesh,
    compiler_params=pltpu.CompilerParams(needs_layout_passes=False),
    out_type=(
        jax.ShapeDtypeStruct((NT, CAP), jnp.int32),
        jax.ShapeDtypeStruct((NT, CAP), jnp.int32),
        jax.ShapeDtypeStruct((NT, 16), jnp.int32),
    ),
    scratch_types=[
        pltpu.VMEM((BAND,), jnp.int32),
        pltpu.VMEM((CAP,), jnp.int32),
        pltpu.VMEM((CAP,), jnp.int32),
        pltpu.VMEM((16,), jnp.int32),
    ],
)
def _sc_compact(keys_hbm, ck_hbm, ci_hbm, cnt_hbm, keys_v, ck_v, ci_v, cnt_v):
    wid = _wid()
    base = wid * BAND
    lane = _lane()
    pltpu.sync_copy(keys_hbm.at[pl.ds(base, BAND)], keys_v)

    def body(j, cnt_vec):
        kv = keys_v[pl.ds(j * 16, 16)]
        mask = kv > 0
        pos = cnt_vec + plsc.cumsum(mask.astype(jnp.int32)) - 1
        okm = mask & (pos < CAP)
        plsc.store_scatter(ck_v, [pos], kv, mask=okm)
        plsc.store_scatter(ci_v, [pos], base + j * 16 + lane, mask=okm)
        return cnt_vec + plsc.all_reduce_population_count(okm)

    cnt_vec = lax.fori_loop(0, BAND // 16, body, jnp.zeros((16,), jnp.int32))
    cnt_v[...] = cnt_vec
    pltpu.sync_copy(ck_v, ck_hbm.at[wid])
    pltpu.sync_copy(ci_v, ci_hbm.at[wid])
    pltpu.sync_copy(cnt_v, cnt_hbm.at[wid])


@functools.partial(
    pl.kernel,
    mesh=_mesh,
    compiler_params=pltpu.CompilerParams(needs_layout_passes=False),
    out_type=jax.ShapeDtypeStruct((N_SEEDS + 16, 16), jnp.float32),
    scratch_types=[
        pltpu.VMEM((NT, CAP), jnp.int32),
        pltpu.VMEM((NT, CAP), jnp.int32),
        pltpu.VMEM((NT, 16), jnp.int32),
        pltpu.VMEM((TOT,), jnp.int32),
        pltpu.VMEM((TOT,), jnp.int32),
        pltpu.VMEM((CAP,), jnp.int32),
        pltpu.VMEM((CAP,), jnp.int32),
        pltpu.VMEM((16, 16), jnp.float32),
        pltpu.VMEM((16,), jnp.float32),
        pltpu.VMEM((16,), jnp.float32),
        pltpu.VMEM((16,), jnp.float32),
        pltpu.VMEM((16,), jnp.float32),
        pltpu.SemaphoreType.DMA,
    ],
)
def _sc_select(ck_hbm, ci_hbm, cnt_hbm, emb_hbm, sig_hbm, seeds_hbm,
               ck_v, ci_v, cnt_v, gk, gi, wbi, wbr, rows, g0, g1, g2, g3, sem):
    wid = _wid()
    lane = _lane()
    zeros16 = jnp.zeros((16,), jnp.int32)

    pltpu.sync_copy(ck_hbm, ck_v)
    pltpu.sync_copy(ci_hbm, ci_v)
    pltpu.sync_copy(cnt_hbm, cnt_v)

    def outer(t, cnt_vec):
        c_t = cnt_v[t][0]

        def inner(j, cv):
            kv = ck_v[t, pl.ds(j * 16, 16)]
            iv = ci_v[t, pl.ds(j * 16, 16)]
            mask = (j * 16 + lane) < c_t
            pos = cv + plsc.cumsum(mask.astype(jnp.int32)) - 1
            plsc.store_scatter(gk, [pos], kv, mask=mask)
            plsc.store_scatter(gi, [pos], iv, mask=mask)
            return cv + plsc.all_reduce_population_count(mask)

        return lax.fori_loop(0, (c_t + 15) // 16, inner, cnt_vec)

    cnt_vec = lax.fori_loop(0, NT, outer, zeros16)
    n_total = cnt_vec[0]
    nv = (n_total + 15) // 16

    def count_ge(thr):
        def b(j, acc):
            kv = gk[pl.ds(j * 16, 16)]
            mm = kv & MBITS
            c = ((j * 16 + lane) < n_total) & (mm >= thr)
            return acc + plsc.all_reduce_population_count(c)

        return lax.fori_loop(0, nv, b, zeros16)[0]

    def bit_step(i, p):
        thr = p | lax.shift_left(jnp.int32(1), 22 - i)
        c1 = count_ge(thr)
        return jnp.where(c1 >= N_SEEDS, thr, p)

    thr_m = lax.fori_loop(0, 23, bit_step, jnp.int32(0))
    n_top = jnp.minimum(jnp.int32(N_SEEDS), n_total)

    def rank_of(mi, idxi):
        def rb(j, acc):
            kv = gk[pl.ds(j * 16, 16)]
            iv = gi[pl.ds(j * 16, 16)]
            mm = kv & MBITS
            hi = ((j * 16 + lane) < n_total) & (
                (mm > mi) | ((mm == mi) & (iv < idxi)))
            return acc + plsc.all_reduce_population_count(hi)

        return lax.fori_loop(0, nv, rb, zeros16)[0]

    def qbody(q, w):
        i = wid + q * NT
        live = i < n_total
        isafe = jnp.full((16,), jnp.minimum(i, TOT - 1), jnp.int32)
        mi = plsc.load_gather(gk, [isafe])[0] & MBITS
        idxi = plsc.load_gather(gi, [isafe])[0]
        live = live & (mi >= thr_m)
        r = lax.cond(live, lambda: rank_of(mi, idxi), lambda: jnp.int32(N_SEEDS))
        win = live & (r < N_SEEDS)
        lane0 = lane == 0
        plsc.store_scatter(wbi, [jnp.full((16,), w, jnp.int32)],
                           jnp.full((16,), idxi, jnp.int32), mask=lane0 & win)
        plsc.store_scatter(wbr, [jnp.full((16,), w, jnp.int32)],
                           jnp.full((16,), r, jnp.int32), mask=lane0 & win)
        return w + jnp.where(win, 1, 0)

    nq = (n_total - wid + NT - 1) // NT
    nq = jnp.maximum(nq, 0)
    w = lax.fori_loop(0, nq, qbody, jnp.int32(0))

    def zrow(j, _):
        plsc.store_scatter(rows, [jnp.full((16,), j, jnp.int32), lane],
                           jnp.zeros((16,), jnp.float32))
        return 0

    lax.fori_loop(0, 16, zrow, 0)
    pad0 = n_top + ((wid - n_top) % NT + NT) % NT

    def zfill(k, _):
        r = pad0 + k * NT

        @pl.when(r < N_SEEDS)
        def _():
            pltpu.sync_copy(rows.at[0], seeds_hbm.at[r])

        return 0

    lax.fori_loop(0, (N_SEEDS + NT - 1) // NT, zfill, 0)

    def chunk(c, _):
        iv = wbi[pl.ds(c * 16, 16)]
        lm = (c * 16 + lane) < w
        ivc = jnp.where(lm, iv, 0)
        pltpu.async_copy(emb_hbm.at[ivc], g0, sem).wait()
        pltpu.async_copy(emb_hbm.at[ivc + NPIX], g1, sem).wait()
        pltpu.async_copy(sig_hbm.at[ivc], g2, sem).wait()
        pltpu.async_copy(sig_hbm.at[ivc + NPIX], g3, sem).wait()

        def col(j):
            return [lane, jnp.full((16,), j, jnp.int32)]

        plsc.store_scatter(rows, col(0),
                           lax.shift_right_logical(ivc, 9).astype(jnp.float32))
        plsc.store_scatter(rows, col(1), (ivc & (W - 1)).astype(jnp.float32))
        plsc.store_scatter(rows, col(2), g0[...])
        plsc.store_scatter(rows, col(3), g1[...])
        plsc.store_scatter(rows, col(4), jnp.exp(g2[...] * 10.0))
        plsc.store_scatter(rows, col(5), jnp.exp(g3[...] * 10.0))
        plsc.store_scatter(rows, col(6), jnp.ones((16,), jnp.float32))

        def wr(k, _):
            pos = c * 16 + k

            @pl.when(pos < w)
            def _():
                rk = plsc.load_gather(
                    wbr, [jnp.full((16,), pos, jnp.int32)])[0]
                pltpu.sync_copy(rows.at[k], seeds_hbm.at[rk])

            return 0

        lax.fori_loop(0, 16, wr, 0)
        return 0

    lax.fori_loop(0, (w + 15) // 16, chunk, 0)


@functools.partial(
    pl.kernel,
    mesh=_mesh,
    compiler_params=pltpu.CompilerParams(needs_layout_passes=False),
    out_type=(
        jax.ShapeDtypeStruct((NPIX,), jnp.float32),
        jax.ShapeDtypeStruct((NPIX,), jnp.float32),
    ),
    scratch_types=[
        pltpu.VMEM((N_SEEDS, 16), jnp.float32),
        pltpu.VMEM((BAND + 128,), jnp.float32),
        pltpu.VMEM((BAND + 128,), jnp.float32),
        pltpu.VMEM((BAND + 128,), jnp.float32),
        pltpu.VMEM((BAND + 128,), jnp.float32),
        pltpu.SemaphoreType.DMA,
    ],
)
def _sc_paint(seeds_hbm, emb_hbm, lab_hbm, sco_hbm, seeds_v, e0v, e1v, mv, lv,
              sem):
    wid = _wid()
    lane = _lane()
    pltpu.sync_copy(seeds_hbm.at[pl.ds(0, N_SEEDS)], seeds_v)
    din = []
    for l in range(ROWS_PER_TILE):
        g = (wid + l * NT) * W
        din.append(pltpu.async_copy(
            emb_hbm.at[pl.ds(g, W)], e0v.at[pl.ds(l * W, W)], sem))
        din.append(pltpu.async_copy(
            emb_hbm.at[pl.ds(NPIX + g, W)], e1v.at[pl.ds(l * W, W)], sem))
    for d in din:
        d.wait()

    def zbody(j, _):
        z = jnp.zeros((16,), jnp.float32)
        mv[pl.ds(j * 16, 16)] = z
        lv[pl.ds(j * 16, 16)] = z
        return 0

    lax.fori_loop(0, BAND // 16, zbody, 0)

    def seed_body(e, _):
        rec = seeds_v[e]
        valid = rec[6]
        py = rec[0].astype(jnp.int32)
        px = rec[1].astype(jnp.int32)
        c0 = rec[2]
        c1 = rec[3]
        s0 = rec[4]
        s1 = rec[5]
        lab = (e + 1).astype(jnp.float32)
        ylo = jnp.maximum(py - WINDOW, 0)
        yhi = jnp.minimum(py + WINDOW, H)
        xlo = jnp.maximum(px - WINDOW, 0)
        xhi = jnp.minimum(px + WINDOW, W)
        y0 = ylo + jnp.mod(wid - ylo, NT)

        cb = lax.shift_right_logical(xlo, 4) * 16

        @pl.when(valid > 0.0)
        def _():
            def one_row(y):
                rowoff = ((y - wid) // NT) * W
                for cch in range(5):
                    base = rowoff + cb + cch * 16
                    bx = cb + cch * 16 + lane
                    mk = (bx >= xlo) & (bx < xhi)
                    a0 = e0v[pl.ds(base, 16)]
                    a1 = e1v[pl.ds(base, 16)]
                    pm = mv[pl.ds(base, 16)]
                    plv = lv[pl.ds(base, 16)]
                    d0 = a0 - c0
                    d1 = a1 - c1
                    pr = jnp.exp(-(d0 * d0 * s0 + d1 * d1 * s1))
                    upd = mk & (pr >= pm)
                    lc = jnp.where(pr >= MASK_THRESH, lab, 0.0)
                    nm = jnp.where(upd, jnp.maximum(pm, pr), pm)
                    nl = jnp.where(upd & (pr > pm), lc,
                                   jnp.where(upd & (pr == pm),
                                             jnp.maximum(plv, lc), plv))
                    mv[pl.ds(base, 16)] = nm
                    lv[pl.ds(base, 16)] = nl

            for k in range(2):
                y = y0 + k * NT

                @pl.when(y < yhi)
                def _():
                    one_row(y)

        return 0

    lax.fori_loop(0, N_SEEDS, seed_body, 0)
    dout = []
    for l in range(ROWS_PER_TILE):
        g = (wid + l * NT) * W
        dout.append(pltpu.async_copy(
            lv.at[pl.ds(l * W, W)], lab_hbm.at[pl.ds(g, W)], sem))
        dout.append(pltpu.async_copy(
            mv.at[pl.ds(l * W, W)], sco_hbm.at[pl.ds(g, W)], sem))
    for d in dout:
        d.wait()


def kernel(fields, sigma, seed_map):
    emb, keys = _tc_stage(fields, seed_map)
    emb_flat = emb.reshape(-1)
    sig_flat = sigma.reshape(-1)
    ck, ci, cnt = _sc_compact(keys.reshape(-1))
    seeds = _sc_select(ck, ci, cnt, emb_flat, sig_flat)
    lab_flat, sco_flat = _sc_paint(seeds, emb_flat)
    return lab_flat.reshape(H, W), sco_flat.reshape(H, W)


# speedup vs baseline: 1.6086x; 1.0460x over previous
import functools

import jax
import jax.numpy as jnp
from jax import lax
from jax.experimental import pallas as pl
from jax.experimental.pallas import tpu as pltpu
from jax.experimental.pallas import tpu_sc as plsc

H = 512
W = 512
WINDOW = 32
N_SEEDS = 256
NEIGH = 5
MIN_SEED = 0.5
MASK_THRESH = 0.5

NT = 32
CAP = 512
TOT = NT * CAP
ROWS_PER_TILE = H // NT
BAND = ROWS_PER_TILE * W
NPIX = H * W
MBITS = 0x7FFFFF

_mesh = plsc.VectorSubcoreMesh(
    core_axis_name="c", subcore_axis_name="s", num_cores=2, num_subcores=16)


def _wid():
    return lax.axis_index("s") * 2 + lax.axis_index("c")


def _lane():
    return lax.iota(jnp.int32, 16)


def _tc_body(fields_ref, seed_ref, emb_ref, keys_ref):
    f = fields_ref[...]
    step = jnp.float32(W * 64.0 / 256.0 / (W - 1))
    col = lax.broadcasted_iota(jnp.int32, (H, W), 1).astype(jnp.float32) * step
    row = lax.broadcasted_iota(jnp.int32, (H, W), 0).astype(jnp.float32) * step
    emb_ref[0] = jnp.tanh(f[0]) + col
    emb_ref[1] = jnp.tanh(f[1]) + row

    s = seed_ref[0]
    ninf = jnp.float32(-jnp.inf)
    rm = s
    for d in range(1, NEIGH + 1):
        left = jnp.concatenate([jnp.full((H, d), ninf), s[:, :-d]], axis=1)
        right = jnp.concatenate([s[:, d:], jnp.full((H, d), ninf)], axis=1)
        rm = jnp.maximum(rm, jnp.maximum(left, right))
    pm = rm
    for d in range(1, NEIGH + 1):
        up = jnp.concatenate([jnp.full((d, W), ninf), rm[:-d, :]], axis=0)
        dn = jnp.concatenate([rm[d:, :], jnp.full((d, W), ninf)], axis=0)
        pm = jnp.maximum(pm, jnp.maximum(up, dn))
    is_peak = (s == pm) & (s > MIN_SEED)
    keys_ref[...] = jnp.where(
        is_peak, lax.bitcast_convert_type(s, jnp.int32), jnp.int32(0))


def _tc_stage(fields, seed_map):
    return pl.pallas_call(
        _tc_body,
        out_shape=(
            jax.ShapeDtypeStruct((2, H, W), jnp.float32),
            jax.ShapeDtypeStruct((H, W), jnp.int32),
        ),
    )(fields, seed_map)


@functools.partial(
    pl.kernel,
    mesh=_mesh,
    compiler_params=pltpu.CompilerParams(needs_layout_passes=False),
    out_type=(
        jax.ShapeDtypeStruct((NT, CAP), jnp.int32),
        jax.ShapeDtypeStruct((NT, CAP), jnp.int32),
        jax.ShapeDtypeStruct((NT, 16), jnp.int32),
    ),
    scratch_types=[
        pltpu.VMEM((BAND,), jnp.int32),
        pltpu.VMEM((CAP,), jnp.int32),
        pltpu.VMEM((CAP,), jnp.int32),
        pltpu.VMEM((16,), jnp.int32),
    ],
)
def _sc_compact(keys_hbm, ck_hbm, ci_hbm, cnt_hbm, keys_v, ck_v, ci_v, cnt_v):
    wid = _wid()
    base = wid * BAND
    lane = _lane()
    pltpu.sync_copy(keys_hbm.at[pl.ds(base, BAND)], keys_v)

    def body(j, cnt_vec):
        kv = keys_v[pl.ds(j * 16, 16)]
        mask = kv > 0
        pos = cnt_vec + plsc.cumsum(mask.astype(jnp.int32)) - 1
        okm = mask & (pos < CAP)
        plsc.store_scatter(ck_v, [pos], kv, mask=okm)
        plsc.store_scatter(ci_v, [pos], base + j * 16 + lane, mask=okm)
        return cnt_vec + plsc.all_reduce_population_count(okm)

    cnt_vec = lax.fori_loop(0, BAND // 16, body, jnp.zeros((16,), jnp.int32))
    cnt_v[...] = cnt_vec
    pltpu.sync_copy(ck_v, ck_hbm.at[wid])
    pltpu.sync_copy(ci_v, ci_hbm.at[wid])
    pltpu.sync_copy(cnt_v, cnt_hbm.at[wid])


@functools.partial(
    pl.kernel,
    mesh=_mesh,
    compiler_params=pltpu.CompilerParams(needs_layout_passes=False),
    out_type=jax.ShapeDtypeStruct((N_SEEDS + 16, 16), jnp.float32),
    scratch_types=[
        pltpu.VMEM((NT, CAP), jnp.int32),
        pltpu.VMEM((NT, CAP), jnp.int32),
        pltpu.VMEM((NT, 16), jnp.int32),
        pltpu.VMEM((TOT,), jnp.int32),
        pltpu.VMEM((TOT,), jnp.int32),
        pltpu.VMEM((TOT,), jnp.int32),
        pltpu.VMEM((TOT,), jnp.int32),
        pltpu.VMEM((CAP,), jnp.int32),
        pltpu.VMEM((CAP,), jnp.int32),
        pltpu.VMEM((16, 16), jnp.float32),
        pltpu.VMEM((16,), jnp.float32),
        pltpu.VMEM((16,), jnp.float32),
        pltpu.VMEM((16,), jnp.float32),
        pltpu.VMEM((16,), jnp.float32),
        pltpu.SemaphoreType.DMA,
    ],
)
def _sc_select(ck_hbm, ci_hbm, cnt_hbm, emb_hbm, sig_hbm, seeds_hbm,
               ck_v, ci_v, cnt_v, gk, gi, sk, si, wbi, wbr, rows,
               g0, g1, g2, g3, sem):
    wid = _wid()
    lane = _lane()
    zeros16 = jnp.zeros((16,), jnp.int32)

    pltpu.sync_copy(ck_hbm, ck_v)
    pltpu.sync_copy(ci_hbm, ci_v)
    pltpu.sync_copy(cnt_hbm, cnt_v)

    def outer(t, cnt_vec):
        c_t = cnt_v[t][0]

        def inner(j, cv):
            kv = ck_v[t, pl.ds(j * 16, 16)]
            iv = ci_v[t, pl.ds(j * 16, 16)]
            mask = (j * 16 + lane) < c_t
            pos = cv + plsc.cumsum(mask.astype(jnp.int32)) - 1
            plsc.store_scatter(gk, [pos], kv, mask=mask)
            plsc.store_scatter(gi, [pos], iv, mask=mask)
            return cv + plsc.all_reduce_population_count(mask)

        return lax.fori_loop(0, (c_t + 15) // 16, inner, cnt_vec)

    cnt_vec = lax.fori_loop(0, NT, outer, zeros16)
    n_total = cnt_vec[0]
    nv = (n_total + 15) // 16

    def count_ge(thr):
        def b(j, acc):
            kv = gk[pl.ds(j * 16, 16)]
            mm = kv & MBITS
            c = ((j * 16 + lane) < n_total) & (mm >= thr)
            return acc + plsc.all_reduce_population_count(c)

        return lax.fori_loop(0, nv, b, zeros16)[0]

    def bit_step(i, p):
        thr = p | lax.shift_left(jnp.int32(1), 22 - i)
        c1 = count_ge(thr)
        return jnp.where(c1 >= N_SEEDS, thr, p)

    p6 = lax.fori_loop(0, 6, bit_step, jnp.int32(0))

    def surv(j, cv):
        kv = gk[pl.ds(j * 16, 16)]
        iv = gi[pl.ds(j * 16, 16)]
        mask = ((j * 16 + lane) < n_total) & ((kv & MBITS) >= p6)
        pos = cv + plsc.cumsum(mask.astype(jnp.int32)) - 1
        plsc.store_scatter(sk, [pos], kv, mask=mask)
        plsc.store_scatter(si, [pos], iv, mask=mask)
        return cv + plsc.all_reduce_population_count(mask)

    s_total = lax.fori_loop(0, nv, surv, zeros16)[0]
    sv = (s_total + 15) // 16

    def count_ge2(thr):
        def b(j, acc):
            kv = sk[pl.ds(j * 16, 16)]
            mm = kv & MBITS
            c = ((j * 16 + lane) < s_total) & (mm >= thr)
            return acc + plsc.all_reduce_population_count(c)

        return lax.fori_loop(0, sv, b, zeros16)[0]

    def bit_step2(i, p):
        thr = p | lax.shift_left(jnp.int32(1), 22 - i)
        c1 = count_ge2(thr)
        return jnp.where(c1 >= N_SEEDS, thr, p)

    thr_m = lax.fori_loop(6, 23, bit_step2, p6)
    n_top = jnp.minimum(jnp.int32(N_SEEDS), n_total)

    def rank_of(mi, idxi):
        def rb(j, acc):
            kv = sk[pl.ds(j * 16, 16)]
            iv = si[pl.ds(j * 16, 16)]
            mm = kv & MBITS
            hi = ((j * 16 + lane) < s_total) & (
                (mm > mi) | ((mm == mi) & (iv < idxi)))
            return acc + plsc.all_reduce_population_count(hi)

        return lax.fori_loop(0, sv, rb, zeros16)[0]

    def qbody(q, w):
        i = wid + q * NT
        live = i < s_total
        isafe = jnp.full((16,), jnp.minimum(i, TOT - 1), jnp.int32)
        mi = plsc.load_gather(sk, [isafe])[0] & MBITS
        idxi = plsc.load_gather(si, [isafe])[0]
        live = live & (mi >= thr_m)
        r = lax.cond(live, lambda: rank_of(mi, idxi), lambda: jnp.int32(N_SEEDS))
        win = live & (r < N_SEEDS)
        lane0 = lane == 0
        plsc.store_scatter(wbi, [jnp.full((16,), w, jnp.int32)],
                           jnp.full((16,), idxi, jnp.int32), mask=lane0 & win)
        plsc.store_scatter(wbr, [jnp.full((16,), w, jnp.int32)],
                           jnp.full((16,), r, jnp.int32), mask=lane0 & win)
        return w + jnp.where(win, 1, 0)

    nq = (s_total - wid + NT - 1) // NT
    nq = jnp.maximum(nq, 0)
    w = lax.fori_loop(0, nq, qbody, jnp.int32(0))

    def zrow(j, _):
        plsc.store_scatter(rows, [jnp.full((16,), j, jnp.int32), lane],
                           jnp.zeros((16,), jnp.float32))
        return 0

    lax.fori_loop(0, 16, zrow, 0)
    pad0 = n_top + ((wid - n_top) % NT + NT) % NT

    def zfill(k, _):
        r = pad0 + k * NT

        @pl.when(r < N_SEEDS)
        def _():
            pltpu.sync_copy(rows.at[0], seeds_hbm.at[r])

        return 0

    lax.fori_loop(0, (N_SEEDS + NT - 1) // NT, zfill, 0)

    def chunk(c, _):
        iv = wbi[pl.ds(c * 16, 16)]
        lm = (c * 16 + lane) < w
        ivc = jnp.where(lm, iv, 0)
        pltpu.async_copy(emb_hbm.at[ivc], g0, sem).wait()
        pltpu.async_copy(emb_hbm.at[ivc + NPIX], g1, sem).wait()
        pltpu.async_copy(sig_hbm.at[ivc], g2, sem).wait()
        pltpu.async_copy(sig_hbm.at[ivc + NPIX], g3, sem).wait()

        def col(j):
            return [lane, jnp.full((16,), j, jnp.int32)]

        plsc.store_scatter(rows, col(0),
                           lax.shift_right_logical(ivc, 9).astype(jnp.float32))
        plsc.store_scatter(rows, col(1), (ivc & (W - 1)).astype(jnp.float32))
        plsc.store_scatter(rows, col(2), g0[...])
        plsc.store_scatter(rows, col(3), g1[...])
        plsc.store_scatter(rows, col(4), jnp.exp(g2[...] * 10.0))
        plsc.store_scatter(rows, col(5), jnp.exp(g3[...] * 10.0))
        plsc.store_scatter(rows, col(6), jnp.ones((16,), jnp.float32))

        def wr(k, _):
            pos = c * 16 + k

            @pl.when(pos < w)
            def _():
                rk = plsc.load_gather(
                    wbr, [jnp.full((16,), pos, jnp.int32)])[0]
                pltpu.sync_copy(rows.at[k], seeds_hbm.at[rk])

            return 0

        lax.fori_loop(0, 16, wr, 0)
        return 0

    lax.fori_loop(0, (w + 15) // 16, chunk, 0)


@functools.partial(
    pl.kernel,
    mesh=_mesh,
    compiler_params=pltpu.CompilerParams(needs_layout_passes=False),
    out_type=(
        jax.ShapeDtypeStruct((NPIX,), jnp.float32),
        jax.ShapeDtypeStruct((NPIX,), jnp.float32),
    ),
    scratch_types=[
        pltpu.VMEM((N_SEEDS, 16), jnp.float32),
        pltpu.VMEM((BAND + 128,), jnp.float32),
        pltpu.VMEM((BAND + 128,), jnp.float32),
        pltpu.VMEM((BAND + 128,), jnp.float32),
        pltpu.VMEM((BAND + 128,), jnp.float32),
        pltpu.SemaphoreType.DMA,
    ],
)
def _sc_paint(seeds_hbm, emb_hbm, lab_hbm, sco_hbm, seeds_v, e0v, e1v, mv, lv,
              sem):
    wid = _wid()
    lane = _lane()
    pltpu.sync_copy(seeds_hbm.at[pl.ds(0, N_SEEDS)], seeds_v)
    din = []
    for l in range(ROWS_PER_TILE):
        g = (wid + l * NT) * W
        din.append(pltpu.async_copy(
            emb_hbm.at[pl.ds(g, W)], e0v.at[pl.ds(l * W, W)], sem))
        din.append(pltpu.async_copy(
            emb_hbm.at[pl.ds(NPIX + g, W)], e1v.at[pl.ds(l * W, W)], sem))
    for d in din:
        d.wait()

    def zbody(j, _):
        z = jnp.zeros((16,), jnp.float32)
        mv[pl.ds(j * 16, 16)] = z
        lv[pl.ds(j * 16, 16)] = z
        return 0

    lax.fori_loop(0, BAND // 16, zbody, 0)

    def seed_body(e, _):
        rec = seeds_v[e]
        valid = rec[6]
        py = rec[0].astype(jnp.int32)
        px = rec[1].astype(jnp.int32)
        c0 = rec[2]
        c1 = rec[3]
        s0 = rec[4]
        s1 = rec[5]
        lab = (e + 1).astype(jnp.float32)
        ylo = jnp.maximum(py - WINDOW, 0)
        yhi = jnp.minimum(py + WINDOW, H)
        xlo = jnp.maximum(px - WINDOW, 0)
        xhi = jnp.minimum(px + WINDOW, W)
        y0 = ylo + jnp.mod(wid - ylo, NT)

        cb = lax.shift_right_logical(xlo, 4) * 16

        @pl.when(valid > 0.0)
        def _():
            def one_row(y):
                rowoff = ((y - wid) // NT) * W
                for cch in range(5):
                    base = rowoff + cb + cch * 16
                    bx = cb + cch * 16 + lane
                    mk = (bx >= xlo) & (bx < xhi)
                    a0 = e0v[pl.ds(base, 16)]
                    a1 = e1v[pl.ds(base, 16)]
                    pm = mv[pl.ds(base, 16)]
                    plv = lv[pl.ds(base, 16)]
                    d0 = a0 - c0
                    d1 = a1 - c1
                    pr = jnp.exp(-(d0 * d0 * s0 + d1 * d1 * s1))
                    upd = mk & (pr >= pm)
                    lc = jnp.where(pr >= MASK_THRESH, lab, 0.0)
                    nm = jnp.where(upd, jnp.maximum(pm, pr), pm)
                    nl = jnp.where(upd & (pr > pm), lc,
                                   jnp.where(upd & (pr == pm),
                                             jnp.maximum(plv, lc), plv))
                    mv[pl.ds(base, 16)] = nm
                    lv[pl.ds(base, 16)] = nl

            for k in range(2):
                y = y0 + k * NT

                @pl.when(y < yhi)
                def _():
                    one_row(y)

        return 0

    lax.fori_loop(0, N_SEEDS, seed_body, 0)
    dout = []
    for l in range(ROWS_PER_TILE):
        g = (wid + l * NT) * W
        dout.append(pltpu.async_copy(
            lv.at[pl.ds(l * W, W)], lab_hbm.at[pl.ds(g, W)], sem))
        dout.append(pltpu.async_copy(
            mv.at[pl.ds(l * W, W)], sco_hbm.at[pl.ds(g, W)], sem))
    for d in dout:
        d.wait()


def kernel(fields, sigma, seed_map):
    emb, keys = _tc_stage(fields, seed_map)
    emb_flat = emb.reshape(-1)
    sig_flat = sigma.reshape(-1)
    ck, ci, cnt = _sc_compact(keys.reshape(-1))
    seeds = _sc_select(ck, ci, cnt, emb_flat, sig_flat)
    lab_flat, sco_flat = _sc_paint(seeds, emb_flat)
    return lab_flat.reshape(H, W), sco_flat.reshape(H, W)


# speedup vs baseline: 1.6415x; 1.0205x over previous
import functools

import jax
import jax.numpy as jnp
from jax import lax
from jax.experimental import pallas as pl
from jax.experimental.pallas import tpu as pltpu
from jax.experimental.pallas import tpu_sc as plsc

H = 512
W = 512
WINDOW = 32
N_SEEDS = 256
NEIGH = 5
MIN_SEED = 0.5
MASK_THRESH = 0.5

NT = 32
CAP = 512
TOT = NT * CAP
ROWS_PER_TILE = H // NT
BAND = ROWS_PER_TILE * W
NPIX = H * W
MBITS = 0x7FFFFF

_mesh = plsc.VectorSubcoreMesh(
    core_axis_name="c", subcore_axis_name="s", num_cores=2, num_subcores=16)


def _wid():
    return lax.axis_index("s") * 2 + lax.axis_index("c")


def _lane():
    return lax.iota(jnp.int32, 16)


def _tc_body(fields_ref, sig_ref, seed_ref, emb_ref, sig_out_ref, keys_ref):
    f = fields_ref[...]
    step = jnp.float32(W * 64.0 / 256.0 / (W - 1))
    col = lax.broadcasted_iota(jnp.int32, (H, W), 1).astype(jnp.float32) * step
    row = lax.broadcasted_iota(jnp.int32, (H, W), 0).astype(jnp.float32) * step
    emb_ref[pl.ds(0, 2048)] = (jnp.tanh(f[0]) + col).reshape(2048, 128)
    emb_ref[pl.ds(2048, 2048)] = (jnp.tanh(f[1]) + row).reshape(2048, 128)
    sg = sig_ref[...]
    sig_out_ref[pl.ds(0, 2048)] = sg[0].reshape(2048, 128)
    sig_out_ref[pl.ds(2048, 2048)] = sg[1].reshape(2048, 128)

    s = seed_ref[0]
    ninf = jnp.float32(-jnp.inf)
    rm = s
    for d in range(1, NEIGH + 1):
        left = jnp.concatenate([jnp.full((H, d), ninf), s[:, :-d]], axis=1)
        right = jnp.concatenate([s[:, d:], jnp.full((H, d), ninf)], axis=1)
        rm = jnp.maximum(rm, jnp.maximum(left, right))
    pm = rm
    for d in range(1, NEIGH + 1):
        up = jnp.concatenate([jnp.full((d, W), ninf), rm[:-d, :]], axis=0)
        dn = jnp.concatenate([rm[d:, :], jnp.full((d, W), ninf)], axis=0)
        pm = jnp.maximum(pm, jnp.maximum(up, dn))
    is_peak = (s == pm) & (s > MIN_SEED)
    keys_ref[...] = jnp.where(
        is_peak, lax.bitcast_convert_type(s, jnp.int32),
        jnp.int32(0)).reshape(2048, 128)


def _tc_stage(fields, sigma, seed_map):
    return pl.pallas_call(
        _tc_body,
        out_shape=(
            jax.ShapeDtypeStruct((4096, 128), jnp.float32),
            jax.ShapeDtypeStruct((4096, 128), jnp.float32),
            jax.ShapeDtypeStruct((2048, 128), jnp.int32),
        ),
    )(fields, sigma, seed_map)


@functools.partial(
    pl.kernel,
    mesh=_mesh,
    compiler_params=pltpu.CompilerParams(needs_layout_passes=False),
    out_type=(
        jax.ShapeDtypeStruct((NT, CAP), jnp.int32),
        jax.ShapeDtypeStruct((NT, CAP), jnp.int32),
        jax.ShapeDtypeStruct((NT, 16), jnp.int32),
    ),
    scratch_types=[
        pltpu.VMEM((BAND,), jnp.int32),
        pltpu.VMEM((CAP,), jnp.int32),
        pltpu.VMEM((CAP,), jnp.int32),
        pltpu.VMEM((16,), jnp.int32),
    ],
)
def _sc_compact(keys_hbm, ck_hbm, ci_hbm, cnt_hbm, keys_v, ck_v, ci_v, cnt_v):
    wid = _wid()
    base = wid * BAND
    lane = _lane()
    pltpu.sync_copy(keys_hbm.at[pl.ds(base, BAND)], keys_v)

    def body(j, cnt_vec):
        kv = keys_v[pl.ds(j * 16, 16)]
        mask = kv > 0
        pos = cnt_vec + plsc.cumsum(mask.astype(jnp.int32)) - 1
        okm = mask & (pos < CAP)
        plsc.store_scatter(ck_v, [pos], kv, mask=okm)
        plsc.store_scatter(ci_v, [pos], base + j * 16 + lane, mask=okm)
        return cnt_vec + plsc.all_reduce_population_count(okm)

    cnt_vec = lax.fori_loop(0, BAND // 16, body, jnp.zeros((16,), jnp.int32))
    cnt_v[...] = cnt_vec
    pltpu.sync_copy(ck_v, ck_hbm.at[wid])
    pltpu.sync_copy(ci_v, ci_hbm.at[wid])
    pltpu.sync_copy(cnt_v, cnt_hbm.at[wid])


@functools.partial(
    pl.kernel,
    mesh=_mesh,
    compiler_params=pltpu.CompilerParams(needs_layout_passes=False),
    out_type=jax.ShapeDtypeStruct((N_SEEDS + 16, 16), jnp.float32),
    scratch_types=[
        pltpu.VMEM((NT, CAP), jnp.int32),
        pltpu.VMEM((NT, CAP), jnp.int32),
        pltpu.VMEM((NT, 16), jnp.int32),
        pltpu.VMEM((TOT,), jnp.int32),
        pltpu.VMEM((TOT,), jnp.int32),
        pltpu.VMEM((TOT,), jnp.int32),
        pltpu.VMEM((TOT,), jnp.int32),
        pltpu.VMEM((CAP,), jnp.int32),
        pltpu.VMEM((CAP,), jnp.int32),
        pltpu.VMEM((16, 16), jnp.float32),
        pltpu.VMEM((16,), jnp.float32),
        pltpu.VMEM((16,), jnp.float32),
        pltpu.VMEM((16,), jnp.float32),
        pltpu.VMEM((16,), jnp.float32),
        pltpu.SemaphoreType.DMA,
    ],
)
def _sc_select(ck_hbm, ci_hbm, cnt_hbm, emb_hbm, sig_hbm, seeds_hbm,
               ck_v, ci_v, cnt_v, gk, gi, sk, si, wbi, wbr, rows,
               g0, g1, g2, g3, sem):
    wid = _wid()
    lane = _lane()
    zeros16 = jnp.zeros((16,), jnp.int32)

    pltpu.sync_copy(ck_hbm, ck_v)
    pltpu.sync_copy(ci_hbm, ci_v)
    pltpu.sync_copy(cnt_hbm, cnt_v)

    def outer(t, cnt_vec):
        c_t = cnt_v[t][0]

        def inner(j, cv):
            kv = ck_v[t, pl.ds(j * 16, 16)]
            iv = ci_v[t, pl.ds(j * 16, 16)]
            mask = (j * 16 + lane) < c_t
            pos = cv + plsc.cumsum(mask.astype(jnp.int32)) - 1
            plsc.store_scatter(gk, [pos], kv, mask=mask)
            plsc.store_scatter(gi, [pos], iv, mask=mask)
            return cv + plsc.all_reduce_population_count(mask)

        return lax.fori_loop(0, (c_t + 15) // 16, inner, cnt_vec)

    cnt_vec = lax.fori_loop(0, NT, outer, zeros16)
    n_total = cnt_vec[0]
    nv = (n_total + 15) // 16

    def count_ge(thr):
        def b(j, acc):
            kv = gk[pl.ds(j * 16, 16)]
            mm = kv & MBITS
            c = ((j * 16 + lane) < n_total) & (mm >= thr)
            return acc + plsc.all_reduce_population_count(c)

        return lax.fori_loop(0, nv, b, zeros16)[0]

    def bit_step(i, p):
        thr = p | lax.shift_left(jnp.int32(1), 22 - i)
        c1 = count_ge(thr)
        return jnp.where(c1 >= N_SEEDS, thr, p)

    p6 = lax.fori_loop(0, 6, bit_step, jnp.int32(0))

    def surv(j, cv):
        kv = gk[pl.ds(j * 16, 16)]
        iv = gi[pl.ds(j * 16, 16)]
        mask = ((j * 16 + lane) < n_total) & ((kv & MBITS) >= p6)
        pos = cv + plsc.cumsum(mask.astype(jnp.int32)) - 1
        plsc.store_scatter(sk, [pos], kv, mask=mask)
        plsc.store_scatter(si, [pos], iv, mask=mask)
        return cv + plsc.all_reduce_population_count(mask)

    s_total = lax.fori_loop(0, nv, surv, zeros16)[0]
    sv = (s_total + 15) // 16

    def count_ge2(thr):
        def b(j, acc):
            kv = sk[pl.ds(j * 16, 16)]
            mm = kv & MBITS
            c = ((j * 16 + lane) < s_total) & (mm >= thr)
            return acc + plsc.all_reduce_population_count(c)

        return lax.fori_loop(0, sv, b, zeros16)[0]

    def bit_step2(i, p):
        thr = p | lax.shift_left(jnp.int32(1), 22 - i)
        c1 = count_ge2(thr)
        return jnp.where(c1 >= N_SEEDS, thr, p)

    thr_m = lax.fori_loop(6, 23, bit_step2, p6)
    n_top = jnp.minimum(jnp.int32(N_SEEDS), n_total)

    def rank_of(mi, idxi):
        def rb(j, acc):
            kv = sk[pl.ds(j * 16, 16)]
            iv = si[pl.ds(j * 16, 16)]
            mm = kv & MBITS
            hi = ((j * 16 + lane) < s_total) & (
                (mm > mi) | ((mm == mi) & (iv < idxi)))
            return acc + plsc.all_reduce_population_count(hi)

        return lax.fori_loop(0, sv, rb, zeros16)[0]

    def qbody(q, w):
        i = wid + q * NT
        live = i < s_total
        isafe = jnp.full((16,), jnp.minimum(i, TOT - 1), jnp.int32)
        mi = plsc.load_gather(sk, [isafe])[0] & MBITS
        idxi = plsc.load_gather(si, [isafe])[0]
        live = live & (mi >= thr_m)
        r = lax.cond(live, lambda: rank_of(mi, idxi), lambda: jnp.int32(N_SEEDS))
        win = live & (r < N_SEEDS)
        lane0 = lane == 0
        plsc.store_scatter(wbi, [jnp.full((16,), w, jnp.int32)],
                           jnp.full((16,), idxi, jnp.int32), mask=lane0 & win)
        plsc.store_scatter(wbr, [jnp.full((16,), w, jnp.int32)],
                           jnp.full((16,), r, jnp.int32), mask=lane0 & win)
        return w + jnp.where(win, 1, 0)

    nq = (s_total - wid + NT - 1) // NT
    nq = jnp.maximum(nq, 0)
    w = lax.fori_loop(0, nq, qbody, jnp.int32(0))

    def zrow(j, _):
        plsc.store_scatter(rows, [jnp.full((16,), j, jnp.int32), lane],
                           jnp.zeros((16,), jnp.float32))
        return 0

    lax.fori_loop(0, 16, zrow, 0)
    pad0 = n_top + ((wid - n_top) % NT + NT) % NT

    def zfill(k, _):
        r = pad0 + k * NT

        @pl.when(r < N_SEEDS)
        def _():
            pltpu.sync_copy(rows.at[0], seeds_hbm.at[r])

        return 0

    lax.fori_loop(0, (N_SEEDS + NT - 1) // NT, zfill, 0)

    def chunk(c, _):
        iv = wbi[pl.ds(c * 16, 16)]
        lm = (c * 16 + lane) < w
        ivc = jnp.where(lm, iv, 0)
        pltpu.async_copy(emb_hbm.at[ivc], g0, sem).wait()
        pltpu.async_copy(emb_hbm.at[ivc + NPIX], g1, sem).wait()
        pltpu.async_copy(sig_hbm.at[ivc], g2, sem).wait()
        pltpu.async_copy(sig_hbm.at[ivc + NPIX], g3, sem).wait()

        def col(j):
            return [lane, jnp.full((16,), j, jnp.int32)]

        plsc.store_scatter(rows, col(0),
                           lax.shift_right_logical(ivc, 9).astype(jnp.float32))
        plsc.store_scatter(rows, col(1), (ivc & (W - 1)).astype(jnp.float32))
        plsc.store_scatter(rows, col(2), g0[...])
        plsc.store_scatter(rows, col(3), g1[...])
        plsc.store_scatter(rows, col(4), jnp.exp(g2[...] * 10.0))
        plsc.store_scatter(rows, col(5), jnp.exp(g3[...] * 10.0))
        plsc.store_scatter(rows, col(6), jnp.ones((16,), jnp.float32))

        def wr(k, _):
            pos = c * 16 + k

            @pl.when(pos < w)
            def _():
                rk = plsc.load_gather(
                    wbr, [jnp.full((16,), pos, jnp.int32)])[0]
                pltpu.sync_copy(rows.at[k], seeds_hbm.at[rk])

            return 0

        lax.fori_loop(0, 16, wr, 0)
        return 0

    lax.fori_loop(0, (w + 15) // 16, chunk, 0)


@functools.partial(
    pl.kernel,
    mesh=_mesh,
    compiler_params=pltpu.CompilerParams(needs_layout_passes=False),
    out_type=(
        jax.ShapeDtypeStruct((NPIX,), jnp.float32),
        jax.ShapeDtypeStruct((NPIX,), jnp.float32),
    ),
    scratch_types=[
        pltpu.VMEM((N_SEEDS, 16), jnp.float32),
        pltpu.VMEM((BAND + 128,), jnp.float32),
        pltpu.VMEM((BAND + 128,), jnp.float32),
        pltpu.VMEM((BAND + 128,), jnp.float32),
        pltpu.VMEM((BAND + 128,), jnp.float32),
        pltpu.SemaphoreType.DMA,
    ],
)
def _sc_paint(seeds_hbm, emb_hbm, lab_hbm, sco_hbm, seeds_v, e0v, e1v, mv, lv,
              sem):
    wid = _wid()
    lane = _lane()
    pltpu.sync_copy(seeds_hbm.at[pl.ds(0, N_SEEDS)], seeds_v)
    din = []
    for l in range(ROWS_PER_TILE):
        g = (wid + l * NT) * W
        din.append(pltpu.async_copy(
            emb_hbm.at[pl.ds(g, W)], e0v.at[pl.ds(l * W, W)], sem))
        din.append(pltpu.async_copy(
            emb_hbm.at[pl.ds(NPIX + g, W)], e1v.at[pl.ds(l * W, W)], sem))
    for d in din:
        d.wait()

    def zbody(j, _):
        z = jnp.zeros((16,), jnp.float32)
        mv[pl.ds(j * 16, 16)] = z
        lv[pl.ds(j * 16, 16)] = z
        return 0

    lax.fori_loop(0, BAND // 16, zbody, 0)

    def seed_body(e, _):
        rec = seeds_v[e]
        valid = rec[6]
        py = rec[0].astype(jnp.int32)
        px = rec[1].astype(jnp.int32)
        c0 = rec[2]
        c1 = rec[3]
        s0 = rec[4]
        s1 = rec[5]
        lab = (e + 1).astype(jnp.float32)
        ylo = jnp.maximum(py - WINDOW, 0)
        yhi = jnp.minimum(py + WINDOW, H)
        xlo = jnp.maximum(px - WINDOW, 0)
        xhi = jnp.minimum(px + WINDOW, W)
        y0 = ylo + jnp.mod(wid - ylo, NT)

        cb = lax.shift_right_logical(xlo, 4) * 16

        @pl.when(valid > 0.0)
        def _():
            def one_row(y):
                rowoff = ((y - wid) // NT) * W
                for cch in range(5):
                    base = rowoff + cb + cch * 16
                    bx = cb + cch * 16 + lane
                    mk = (bx >= xlo) & (bx < xhi)
                    a0 = e0v[pl.ds(base, 16)]
                    a1 = e1v[pl.ds(base, 16)]
                    pm = mv[pl.ds(base, 16)]
                    plv = lv[pl.ds(base, 16)]
                    d0 = a0 - c0
                    d1 = a1 - c1
                    pr = jnp.exp(-(d0 * d0 * s0 + d1 * d1 * s1))
                    upd = mk & (pr >= pm)
                    lc = jnp.where(pr >= MASK_THRESH, lab, 0.0)
                    nm = jnp.where(upd, jnp.maximum(pm, pr), pm)
                    nl = jnp.where(upd & (pr > pm), lc,
                                   jnp.where(upd & (pr == pm),
                                             jnp.maximum(plv, lc), plv))
                    mv[pl.ds(base, 16)] = nm
                    lv[pl.ds(base, 16)] = nl

            for k in range(2):
                y = y0 + k * NT

                @pl.when(y < yhi)
                def _():
                    one_row(y)

        return 0

    lax.fori_loop(0, N_SEEDS, seed_body, 0)
    dout = []
    for l in range(ROWS_PER_TILE):
        g = (wid + l * NT) * W
        dout.append(pltpu.async_copy(
            lv.at[pl.ds(l * W, W)], lab_hbm.at[pl.ds(g, W)], sem))
        dout.append(pltpu.async_copy(
            mv.at[pl.ds(l * W, W)], sco_hbm.at[pl.ds(g, W)], sem))
    for d in dout:
        d.wait()


def kernel(fields, sigma, seed_map):
    emb, sig, keys = _tc_stage(fields, sigma, seed_map)
    emb_flat = emb.reshape(-1)
    sig_flat = sig.reshape(-1)
    ck, ci, cnt = _sc_compact(keys.reshape(-1))
    seeds = _sc_select(ck, ci, cnt, emb_flat, sig_flat)
    lab_flat, sco_flat = _sc_paint(seeds, emb_flat)
    return lab_flat.reshape(H, W), sco_flat.reshape(H, W)
